# baseline (speedup 1.0000x reference)
import jax
import jax.numpy as jnp
from jax.experimental import pallas as pl
from jax.experimental.pallas import tpu as pltpu
from jax.experimental.pallas import tpu_sc as plsc

NUM_K = 8192
DIM = 64
BATCH = 8
TOKENS_PER_ROW = 1024
N_TOKENS = BATCH * TOKENS_PER_ROW
KBLK = 1024
TBLK = 1024


def _block_argmin(dist, bmin, k):
    iota = jax.lax.broadcasted_iota(
        jnp.int32, dist.shape, 1).astype(jnp.float32)
    barg_f = jnp.min(jnp.where(dist == bmin, iota, jnp.float32(2.0 ** 30)),
                     axis=1, keepdims=True)
    return barg_f.astype(jnp.int32) + k * KBLK


def _argmin_body(u_ref, wt2_ref, a_ref, b_ref, min_ref, arg_ref, acc_ref):
    t = pl.program_id(0)
    k = pl.program_id(1)
    nk = pl.num_programs(1)
    c2 = jax.lax.dot_general(
        u_ref[...], wt2_ref[...], (((1,), (0,)), ((), ())),
        preferred_element_type=jnp.float32)
    a = a_ref[...]

    fast = jnp.min(a) >= 32.0

    @pl.when(fast)
    def _():
        bmax = jnp.max(c2, axis=1, keepdims=True)

        @pl.when(t == 0)
        def _():
            bmin_d = a - bmax
            dist = a - c2
            barg = _block_argmin(dist, bmin_d, k)

            @pl.when(k == 0)
            def _():
                arg_ref[...] = barg

            @pl.when(k != 0)
            def _():
                run_d = a - acc_ref[...]
                arg_ref[...] = jnp.where(bmin_d < run_d, barg, arg_ref[...])

        @pl.when(k == 0)
        def _():
            acc_ref[...] = bmax

        @pl.when(k != 0)
        def _():
            acc_ref[...] = jnp.maximum(acc_ref[...], bmax)

    @pl.when(jnp.logical_not(fast))
    def _():
        dist = (a + b_ref[...]) - c2
        bmin = jnp.min(dist, axis=1, keepdims=True)

        @pl.when(t == 0)
        def _():
            barg = _block_argmin(dist, bmin, k)

            @pl.when(k == 0)
            def _():
                arg_ref[...] = barg

            @pl.when(k != 0)
            def _():
                arg_ref[...] = jnp.where(bmin < acc_ref[...], barg,
                                         arg_ref[...])

        @pl.when(k == 0)
        def _():
            acc_ref[...] = bmin

        @pl.when(k != 0)
        def _():
            acc_ref[...] = jnp.minimum(acc_ref[...], bmin)

    @pl.when(k == nk - 1)
    def _():
        @pl.when(fast)
        def _():
            min_ref[...] = a - acc_ref[...]

        @pl.when(jnp.logical_not(fast))
        def _():
            min_ref[...] = acc_ref[...]


def _sort_body(key_ref, enc_ref, out_ref):
    keys = key_ref[...]
    vals = enc_ref[...]
    n = keys.shape[1]
    pos = jax.lax.broadcasted_iota(jnp.int32, keys.shape, 1)
    i = pos

    def partner(x, j, lower):
        fwd = jnp.roll(x, -j, axis=1)
        bwd = jnp.roll(x, j, axis=1)
        return jnp.where(lower, fwd, bwd)

    k = 2
    while k <= n:
        j = k // 2
        while j >= 1:
            lower = (i & j) == 0
            pk = partner(keys, j, lower)
            pp = partner(pos, j, lower)
            pv = partner(vals, j, lower)
            asc = (i & k) == 0
            lt = (pk < keys) | ((pk == keys) & (pp < pos))
            take = lt == (lower == asc)
            keys = jnp.where(take, pk, keys)
            pos = jnp.where(take, pp, pos)
            vals = jnp.where(take, pv, vals)
            j //= 2
        k *= 2
    out_ref[...] = vals


def _sc_gather(weight, idx2d, n_rows):
    mesh = plsc.VectorSubcoreMesh(core_axis_name="core",
                                  subcore_axis_name="subcore")
    window = 128
    wpad = jnp.pad(weight, ((0, 0), (0, 128 - DIM)))

    @pl.kernel(out_type=jax.ShapeDtypeStruct((n_rows, 128), weight.dtype),
               mesh=mesh)
    def kern(x_hbm, i_hbm, o_hbm):
        def body(i_vmem, o_vmem):
            pltpu.sync_copy(x_hbm.at[i_vmem.at[0]], o_vmem)

        pltpu.emit_pipeline(
            body,
            grid=(n_rows // window,),
            in_specs=[pl.BlockSpec((1, window), lambda i: (0, i))],
            out_specs=[pl.BlockSpec((window, 128), lambda i: (i, 0))],
            core_axis_name=("core", "subcore"),
            dimension_semantics=(pltpu.PARALLEL,),
        )(i_hbm, o_hbm)

    return kern(wpad, idx2d)[:, :DIM]


def kernel(inputs, weight):
    input_shape = inputs.shape
    flat = inputs.reshape(-1, DIM)
    wt2 = 2.0 * weight.T
    a = jnp.sum(flat ** 2, axis=1, keepdims=True)
    b = jnp.sum(weight ** 2, axis=1)[None, :]

    grid = (N_TOKENS // TBLK, NUM_K // KBLK)
    mins, args = pl.pallas_call(
        _argmin_body,
        grid=grid,
        in_specs=[
            pl.BlockSpec((TBLK, DIM), lambda t, k: (t, 0)),
            pl.BlockSpec((DIM, KBLK), lambda t, k: (0, k)),
            pl.BlockSpec((TBLK, 1), lambda t, k: (t, 0)),
            pl.BlockSpec((1, KBLK), lambda t, k: (0, k)),
        ],
        out_specs=[
            pl.BlockSpec((TBLK, 1), lambda t, k: (t, 0)),
            pl.BlockSpec((TBLK, 1), lambda t, k: (0, 0)),
        ],
        out_shape=[
            jax.ShapeDtypeStruct((N_TOKENS, 1), jnp.float32),
            jax.ShapeDtypeStruct((TBLK, 1), jnp.int32),
        ],
        scratch_shapes=[pltpu.VMEM((TBLK, 1), jnp.float32)],
        compiler_params=pltpu.CompilerParams(
            dimension_semantics=("parallel", "arbitrary")),
    )(flat, wt2, a, b)

    keys = mins[:, 0].reshape(BATCH, TOKENS_PER_ROW)
    enc0 = jnp.broadcast_to(args[:TOKENS_PER_ROW, 0][None, :],
                            (BATCH, TOKENS_PER_ROW))

    final_idx = pl.pallas_call(
        _sort_body,
        in_specs=[
            pl.BlockSpec((BATCH, TOKENS_PER_ROW), lambda: (0, 0)),
            pl.BlockSpec((BATCH, TOKENS_PER_ROW), lambda: (0, 0)),
        ],
        out_specs=pl.BlockSpec((BATCH, TOKENS_PER_ROW), lambda: (0, 0)),
        out_shape=jax.ShapeDtypeStruct((BATCH, TOKENS_PER_ROW), jnp.int32),
    )(keys, enc0)

    gathered = _sc_gather(weight, final_idx.reshape(1, N_TOKENS), N_TOKENS)
    return gathered.reshape(input_shape)


# speedup vs baseline: 1.0102x; 1.0102x over previous
import jax
import jax.numpy as jnp
from jax.experimental import pallas as pl
from jax.experimental.pallas import tpu as pltpu
from jax.experimental.pallas import tpu_sc as plsc

NUM_K = 8192
DIM = 64
BATCH = 8
TOKENS_PER_ROW = 1024
N_TOKENS = BATCH * TOKENS_PER_ROW
KBLK = 1024
TBLK = 1024


def _block_argmin(dist, bmin, k):
    iota = jax.lax.broadcasted_iota(
        jnp.int32, dist.shape, 1).astype(jnp.float32)
    barg_f = jnp.min(jnp.where(dist == bmin, iota, jnp.float32(2.0 ** 30)),
                     axis=1, keepdims=True)
    return barg_f.astype(jnp.int32) + k * KBLK


def _argmin_body(flag_ref, u_ref, wt2_ref, a_ref, b_ref, min_ref, arg_ref,
                 acc_ref):
    t = pl.program_id(0)
    k = pl.program_id(1)
    nk = pl.num_programs(1)
    c2 = jax.lax.dot_general(
        u_ref[...], wt2_ref[...], (((1,), (0,)), ((), ())),
        preferred_element_type=jnp.float32)
    a = a_ref[...]

    fast = flag_ref[t] == 1

    @pl.when(fast)
    def _():
        bmax = jnp.max(c2, axis=1, keepdims=True)

        @pl.when(t == 0)
        def _():
            bmin_d = a - bmax
            dist = a - c2
            barg = _block_argmin(dist, bmin_d, k)

            @pl.when(k == 0)
            def _():
                arg_ref[...] = barg

            @pl.when(k != 0)
            def _():
                run_d = a - acc_ref[...]
                arg_ref[...] = jnp.where(bmin_d < run_d, barg, arg_ref[...])

        @pl.when(k == 0)
        def _():
            acc_ref[...] = bmax

        @pl.when(k != 0)
        def _():
            acc_ref[...] = jnp.maximum(acc_ref[...], bmax)

    @pl.when(jnp.logical_not(fast))
    def _():
        dist = (a + b_ref[...]) - c2
        bmin = jnp.min(dist, axis=1, keepdims=True)

        @pl.when(t == 0)
        def _():
            barg = _block_argmin(dist, bmin, k)

            @pl.when(k == 0)
            def _():
                arg_ref[...] = barg

            @pl.when(k != 0)
            def _():
                arg_ref[...] = jnp.where(bmin < acc_ref[...], barg,
                                         arg_ref[...])

        @pl.when(k == 0)
        def _():
            acc_ref[...] = bmin

        @pl.when(k != 0)
        def _():
            acc_ref[...] = jnp.minimum(acc_ref[...], bmin)

    @pl.when(k == nk - 1)
    def _():
        @pl.when(fast)
        def _():
            min_ref[...] = a - acc_ref[...]

        @pl.when(jnp.logical_not(fast))
        def _():
            min_ref[...] = acc_ref[...]


def _sort_body(key_ref, enc_ref, out_ref):
    keys = key_ref[...]
    vals = enc_ref[...]
    n = keys.shape[1]
    pos = jax.lax.broadcasted_iota(jnp.int32, keys.shape, 1)
    i = pos

    def partner(x, j, lower):
        fwd = jnp.roll(x, -j, axis=1)
        bwd = jnp.roll(x, j, axis=1)
        return jnp.where(lower, fwd, bwd)

    k = 2
    while k <= n:
        j = k // 2
        while j >= 1:
            lower = (i & j) == 0
            pk = partner(keys, j, lower)
            pp = partner(pos, j, lower)
            pv = partner(vals, j, lower)
            asc = (i & k) == 0
            lt = (pk < keys) | ((pk == keys) & (pp < pos))
            take = lt == (lower == asc)
            keys = jnp.where(take, pk, keys)
            pos = jnp.where(take, pp, pos)
            vals = jnp.where(take, pv, vals)
            j //= 2
        k *= 2
    out_ref[...] = vals


def _sc_gather(weight, idx2d, n_rows):
    mesh = plsc.VectorSubcoreMesh(core_axis_name="core",
                                  subcore_axis_name="subcore")
    window = 128
    wpad = jnp.pad(weight, ((0, 0), (0, 128 - DIM)))

    @pl.kernel(out_type=jax.ShapeDtypeStruct((n_rows, 128), weight.dtype),
               mesh=mesh)
    def kern(x_hbm, i_hbm, o_hbm):
        def body(i_vmem, o_vmem):
            pltpu.sync_copy(x_hbm.at[i_vmem.at[0]], o_vmem)

        pltpu.emit_pipeline(
            body,
            grid=(n_rows // window,),
            in_specs=[pl.BlockSpec((1, window), lambda i: (0, i))],
            out_specs=[pl.BlockSpec((window, 128), lambda i: (i, 0))],
            core_axis_name=("core", "subcore"),
            dimension_semantics=(pltpu.PARALLEL,),
        )(i_hbm, o_hbm)

    return kern(wpad, idx2d)[:, :DIM]


def kernel(inputs, weight):
    input_shape = inputs.shape
    flat = inputs.reshape(-1, DIM)
    wt2 = 2.0 * weight.T
    a = jnp.sum(flat ** 2, axis=1, keepdims=True)
    b = jnp.sum(weight ** 2, axis=1)[None, :]
    flags = ((jnp.min(a.reshape(N_TOKENS // TBLK, TBLK), axis=1) >= 16.0)
             & (jnp.max(b) < 2.0 ** -20)).astype(jnp.int32)

    grid = (N_TOKENS // TBLK, NUM_K // KBLK)
    mins, args = pl.pallas_call(
        _argmin_body,
        grid=grid,
        in_specs=[
            pl.BlockSpec(memory_space=pltpu.SMEM),
            pl.BlockSpec((TBLK, DIM), lambda t, k: (t, 0)),
            pl.BlockSpec((DIM, KBLK), lambda t, k: (0, k)),
            pl.BlockSpec((TBLK, 1), lambda t, k: (t, 0)),
            pl.BlockSpec((1, KBLK), lambda t, k: (0, k)),
        ],
        out_specs=[
            pl.BlockSpec((TBLK, 1), lambda t, k: (t, 0)),
            pl.BlockSpec((TBLK, 1), lambda t, k: (0, 0)),
        ],
        out_shape=[
            jax.ShapeDtypeStruct((N_TOKENS, 1), jnp.float32),
            jax.ShapeDtypeStruct((TBLK, 1), jnp.int32),
        ],
        scratch_shapes=[pltpu.VMEM((TBLK, 1), jnp.float32)],
        compiler_params=pltpu.CompilerParams(
            dimension_semantics=("parallel", "arbitrary")),
    )(flags, flat, wt2, a, b)

    keys = mins[:, 0].reshape(BATCH, TOKENS_PER_ROW)
    enc0 = jnp.broadcast_to(args[:TOKENS_PER_ROW, 0][None, :],
                            (BATCH, TOKENS_PER_ROW))

    final_idx = pl.pallas_call(
        _sort_body,
        in_specs=[
            pl.BlockSpec((BATCH, TOKENS_PER_ROW), lambda: (0, 0)),
            pl.BlockSpec((BATCH, TOKENS_PER_ROW), lambda: (0, 0)),
        ],
        out_specs=pl.BlockSpec((BATCH, TOKENS_PER_ROW), lambda: (0, 0)),
        out_shape=jax.ShapeDtypeStruct((BATCH, TOKENS_PER_ROW), jnp.int32),
    )(keys, enc0)

    gathered = _sc_gather(weight, final_idx.reshape(1, N_TOKENS), N_TOKENS)
    return gathered.reshape(input_shape)


# speedup vs baseline: 1.0560x; 1.0454x over previous
import jax
import jax.numpy as jnp
from jax.experimental import pallas as pl
from jax.experimental.pallas import tpu as pltpu
from jax.experimental.pallas import tpu_sc as plsc

NUM_K = 8192
DIM = 64
BATCH = 8
TOKENS_PER_ROW = 1024
N_TOKENS = BATCH * TOKENS_PER_ROW
KBLK = 1024
TBLK = 1024


def _block_argmin(dist, bmin, k):
    iota = jax.lax.broadcasted_iota(
        jnp.int32, dist.shape, 1).astype(jnp.float32)
    barg_f = jnp.min(jnp.where(dist == bmin, iota, jnp.float32(2.0 ** 30)),
                     axis=1, keepdims=True)
    return barg_f.astype(jnp.int32) + k * KBLK


def _fast_body(u_ref, wt2_ref, a_ref, min_ref, arg_ref):
    t = pl.program_id(0)
    k = pl.program_id(1)
    nk = pl.num_programs(1)
    c2 = jax.lax.dot_general(
        u_ref[...], wt2_ref[...], (((1,), (0,)), ((), ())),
        preferred_element_type=jnp.float32)
    a = a_ref[...]
    bmax = jnp.max(c2, axis=1, keepdims=True)

    @pl.when(t == 0)
    def _():
        bmin_d = a - bmax
        dist = a - c2
        barg = _block_argmin(dist, bmin_d, k)

        @pl.when(k == 0)
        def _():
            arg_ref[...] = barg

        @pl.when(k != 0)
        def _():
            run_d = a - min_ref[...]
            arg_ref[...] = jnp.where(bmin_d < run_d, barg, arg_ref[...])

    @pl.when(k == 0)
    def _():
        min_ref[...] = bmax

    @pl.when(k != 0)
    def _():
        min_ref[...] = jnp.maximum(min_ref[...], bmax)

    @pl.when(k == nk - 1)
    def _():
        min_ref[...] = a - min_ref[...]


def _exact_body(u_ref, wt2_ref, a_ref, b_ref, min_ref, arg_ref):
    t = pl.program_id(0)
    k = pl.program_id(1)
    c2 = jax.lax.dot_general(
        u_ref[...], wt2_ref[...], (((1,), (0,)), ((), ())),
        preferred_element_type=jnp.float32)
    dist = (a_ref[...] + b_ref[...]) - c2
    bmin = jnp.min(dist, axis=1, keepdims=True)

    @pl.when(t == 0)
    def _():
        barg = _block_argmin(dist, bmin, k)

        @pl.when(k == 0)
        def _():
            arg_ref[...] = barg

        @pl.when(k != 0)
        def _():
            arg_ref[...] = jnp.where(bmin < min_ref[...], barg, arg_ref[...])

    @pl.when(k == 0)
    def _():
        min_ref[...] = bmin

    @pl.when(k != 0)
    def _():
        min_ref[...] = jnp.minimum(min_ref[...], bmin)


def _sort_body(key_ref, enc_ref, out_ref):
    keys = key_ref[...]
    vals = enc_ref[...]
    n = keys.shape[1]
    pos = jax.lax.broadcasted_iota(jnp.int32, keys.shape, 1)
    i = pos

    def partner(x, j, lower):
        fwd = jnp.roll(x, -j, axis=1)
        bwd = jnp.roll(x, j, axis=1)
        return jnp.where(lower, fwd, bwd)

    k = 2
    while k <= n:
        j = k // 2
        while j >= 1:
            lower = (i & j) == 0
            pk = partner(keys, j, lower)
            pp = partner(pos, j, lower)
            pv = partner(vals, j, lower)
            asc = (i & k) == 0
            lt = (pk < keys) | ((pk == keys) & (pp < pos))
            take = lt == (lower == asc)
            keys = jnp.where(take, pk, keys)
            pos = jnp.where(take, pp, pos)
            vals = jnp.where(take, pv, vals)
            j //= 2
        k *= 2
    out_ref[...] = vals


def _sc_gather(weight, idx2d, n_rows):
    mesh = plsc.VectorSubcoreMesh(core_axis_name="core",
                                  subcore_axis_name="subcore")
    window = 128
    wpad = jnp.pad(weight, ((0, 0), (0, 128 - DIM)))

    @pl.kernel(out_type=jax.ShapeDtypeStruct((n_rows, 128), weight.dtype),
               mesh=mesh)
    def kern(x_hbm, i_hbm, o_hbm):
        def body(i_vmem, o_vmem):
            pltpu.sync_copy(x_hbm.at[i_vmem.at[0]], o_vmem)

        pltpu.emit_pipeline(
            body,
            grid=(n_rows // window,),
            in_specs=[pl.BlockSpec((1, window), lambda i: (0, i))],
            out_specs=[pl.BlockSpec((window, 128), lambda i: (i, 0))],
            core_axis_name=("core", "subcore"),
            dimension_semantics=(pltpu.PARALLEL,),
        )(i_hbm, o_hbm)

    return kern(wpad, idx2d)[:, :DIM]


def kernel(inputs, weight):
    input_shape = inputs.shape
    flat = inputs.reshape(-1, DIM)
    wt2 = 2.0 * weight.T
    a = jnp.sum(flat ** 2, axis=1, keepdims=True)
    b = jnp.sum(weight ** 2, axis=1)[None, :]
    all_fast = (jnp.min(a) >= 16.0) & (jnp.max(b) < 2.0 ** -20)

    grid = (N_TOKENS // TBLK, NUM_K // KBLK)
    out_specs = [
        pl.BlockSpec((TBLK, 1), lambda t, k: (t, 0)),
        pl.BlockSpec((TBLK, 1), lambda t, k: (0, 0)),
    ]
    out_shape = [
        jax.ShapeDtypeStruct((N_TOKENS, 1), jnp.float32),
        jax.ShapeDtypeStruct((TBLK, 1), jnp.int32),
    ]
    cparams = pltpu.CompilerParams(
        dimension_semantics=("parallel", "arbitrary"))

    def _run_fast(ops):
        flat, wt2, a, _ = ops
        return pl.pallas_call(
            _fast_body,
            grid=grid,
            in_specs=[
                pl.BlockSpec((TBLK, DIM), lambda t, k: (t, 0)),
                pl.BlockSpec((DIM, KBLK), lambda t, k: (0, k)),
                pl.BlockSpec((TBLK, 1), lambda t, k: (t, 0)),
            ],
            out_specs=out_specs,
            out_shape=out_shape,
            compiler_params=cparams,
        )(flat, wt2, a)

    def _run_exact(ops):
        flat, wt2, a, b = ops
        return pl.pallas_call(
            _exact_body,
            grid=grid,
            in_specs=[
                pl.BlockSpec((TBLK, DIM), lambda t, k: (t, 0)),
                pl.BlockSpec((DIM, KBLK), lambda t, k: (0, k)),
                pl.BlockSpec((TBLK, 1), lambda t, k: (t, 0)),
                pl.BlockSpec((1, KBLK), lambda t, k: (0, k)),
            ],
            out_specs=out_specs,
            out_shape=out_shape,
            compiler_params=cparams,
        )(flat, wt2, a, b)

    mins, args = jax.lax.cond(all_fast, _run_fast, _run_exact,
                              (flat, wt2, a, b))

    keys = mins[:, 0].reshape(BATCH, TOKENS_PER_ROW)
    enc0 = jnp.broadcast_to(args[:TOKENS_PER_ROW, 0][None, :],
                            (BATCH, TOKENS_PER_ROW))

    final_idx = pl.pallas_call(
        _sort_body,
        in_specs=[
            pl.BlockSpec((BATCH, TOKENS_PER_ROW), lambda: (0, 0)),
            pl.BlockSpec((BATCH, TOKENS_PER_ROW), lambda: (0, 0)),
        ],
        out_specs=pl.BlockSpec((BATCH, TOKENS_PER_ROW), lambda: (0, 0)),
        out_shape=jax.ShapeDtypeStruct((BATCH, TOKENS_PER_ROW), jnp.int32),
    )(keys, enc0)

    gathered = _sc_gather(weight, final_idx.reshape(1, N_TOKENS), N_TOKENS)
    return gathered.reshape(input_shape)


# speedup vs baseline: 1.0657x; 1.0092x over previous
import jax
import jax.numpy as jnp
from jax.experimental import pallas as pl
from jax.experimental.pallas import tpu as pltpu
from jax.experimental.pallas import tpu_sc as plsc

NUM_K = 8192
DIM = 64
BATCH = 8
TOKENS_PER_ROW = 1024
N_TOKENS = BATCH * TOKENS_PER_ROW
KBLK = 1024
TBLK = 1024


def _block_argmin(dist, bmin, k):
    iota = jax.lax.broadcasted_iota(
        jnp.int32, dist.shape, 1).astype(jnp.float32)
    barg_f = jnp.min(jnp.where(dist == bmin, iota, jnp.float32(2.0 ** 30)),
                     axis=1, keepdims=True)
    return barg_f.astype(jnp.int32) + k * KBLK


def _fast_body(u_ref, wt2_ref, a_ref, min_ref, arg_ref):
    t = pl.program_id(0)
    k = pl.program_id(1)
    nk = pl.num_programs(1)
    c2 = jax.lax.dot_general(
        u_ref[...], wt2_ref[...], (((1,), (0,)), ((), ())),
        preferred_element_type=jnp.float32)
    a = a_ref[...]
    bmax = jnp.max(c2, axis=1, keepdims=True)

    @pl.when(t == 0)
    def _():
        bmin_d = a - bmax
        dist = a - c2
        barg = _block_argmin(dist, bmin_d, k)

        @pl.when(k == 0)
        def _():
            arg_ref[...] = barg

        @pl.when(k != 0)
        def _():
            run_d = a - min_ref[...]
            arg_ref[...] = jnp.where(bmin_d < run_d, barg, arg_ref[...])

    @pl.when(k == 0)
    def _():
        min_ref[...] = bmax

    @pl.when(k != 0)
    def _():
        min_ref[...] = jnp.maximum(min_ref[...], bmax)

    @pl.when(k == nk - 1)
    def _():
        min_ref[...] = a - min_ref[...]


def _exact_body(u_ref, wt2_ref, a_ref, b_ref, min_ref, arg_ref):
    t = pl.program_id(0)
    k = pl.program_id(1)
    c2 = jax.lax.dot_general(
        u_ref[...], wt2_ref[...], (((1,), (0,)), ((), ())),
        preferred_element_type=jnp.float32)
    dist = (a_ref[...] + b_ref[...]) - c2
    bmin = jnp.min(dist, axis=1, keepdims=True)

    @pl.when(t == 0)
    def _():
        barg = _block_argmin(dist, bmin, k)

        @pl.when(k == 0)
        def _():
            arg_ref[...] = barg

        @pl.when(k != 0)
        def _():
            arg_ref[...] = jnp.where(bmin < min_ref[...], barg, arg_ref[...])

    @pl.when(k == 0)
    def _():
        min_ref[...] = bmin

    @pl.when(k != 0)
    def _():
        min_ref[...] = jnp.minimum(min_ref[...], bmin)


def _sort_body(key_ref, enc_ref, out_ref):
    keys = key_ref[...]
    vals = enc_ref[...]
    n = keys.shape[1]
    pos = jax.lax.broadcasted_iota(jnp.int32, keys.shape, 1)
    i = pos

    def partner(x, j, lower):
        fwd = jnp.roll(x, -j, axis=1)
        bwd = jnp.roll(x, j, axis=1)
        return jnp.where(lower, fwd, bwd)

    k = 2
    while k <= n:
        j = k // 2
        while j >= 1:
            lower = (i & j) == 0
            pk = partner(keys, j, lower)
            pp = partner(pos, j, lower)
            pv = partner(vals, j, lower)
            asc = (i & k) == 0
            lt = (pk < keys) | ((pk == keys) & (pp < pos))
            take = lt == (lower == asc)
            keys = jnp.where(take, pk, keys)
            pos = jnp.where(take, pp, pos)
            vals = jnp.where(take, pv, vals)
            j //= 2
        k *= 2
    out_ref[...] = vals


def _sc_gather(weight, idx2d, n_rows):
    mesh = plsc.VectorSubcoreMesh(core_axis_name="core",
                                  subcore_axis_name="subcore")
    window = 256
    wpad = jnp.pad(weight, ((0, 0), (0, 128 - DIM)))

    @pl.kernel(out_type=jax.ShapeDtypeStruct((n_rows, 128), weight.dtype),
               mesh=mesh)
    def kern(x_hbm, i_hbm, o_hbm):
        def body(i_vmem, o_vmem):
            pltpu.sync_copy(x_hbm.at[i_vmem.at[0]], o_vmem)

        pltpu.emit_pipeline(
            body,
            grid=(n_rows // window,),
            in_specs=[pl.BlockSpec((1, window), lambda i: (0, i))],
            out_specs=[pl.BlockSpec((window, 128), lambda i: (i, 0))],
            core_axis_name=("core", "subcore"),
            dimension_semantics=(pltpu.PARALLEL,),
        )(i_hbm, o_hbm)

    return kern(wpad, idx2d)[:, :DIM]


def kernel(inputs, weight):
    input_shape = inputs.shape
    flat = inputs.reshape(-1, DIM)
    wt2 = 2.0 * weight.T
    a = jnp.sum(flat ** 2, axis=1, keepdims=True)
    b = jnp.sum(weight ** 2, axis=1)[None, :]
    all_fast = (jnp.min(a) >= 16.0) & (jnp.max(b) < 2.0 ** -20)

    grid = (N_TOKENS // TBLK, NUM_K // KBLK)
    out_specs = [
        pl.BlockSpec((TBLK, 1), lambda t, k: (t, 0)),
        pl.BlockSpec((TBLK, 1), lambda t, k: (0, 0)),
    ]
    out_shape = [
        jax.ShapeDtypeStruct((N_TOKENS, 1), jnp.float32),
        jax.ShapeDtypeStruct((TBLK, 1), jnp.int32),
    ]
    cparams = pltpu.CompilerParams(
        dimension_semantics=("parallel", "arbitrary"))

    def _run_fast(ops):
        flat, wt2, a, _ = ops
        return pl.pallas_call(
            _fast_body,
            grid=grid,
            in_specs=[
                pl.BlockSpec((TBLK, DIM), lambda t, k: (t, 0)),
                pl.BlockSpec((DIM, KBLK), lambda t, k: (0, k)),
                pl.BlockSpec((TBLK, 1), lambda t, k: (t, 0)),
            ],
            out_specs=out_specs,
            out_shape=out_shape,
            compiler_params=cparams,
        )(flat, wt2, a)

    def _run_exact(ops):
        flat, wt2, a, b = ops
        return pl.pallas_call(
            _exact_body,
            grid=grid,
            in_specs=[
                pl.BlockSpec((TBLK, DIM), lambda t, k: (t, 0)),
                pl.BlockSpec((DIM, KBLK), lambda t, k: (0, k)),
                pl.BlockSpec((TBLK, 1), lambda t, k: (t, 0)),
                pl.BlockSpec((1, KBLK), lambda t, k: (0, k)),
            ],
            out_specs=out_specs,
            out_shape=out_shape,
            compiler_params=cparams,
        )(flat, wt2, a, b)

    mins, args = jax.lax.cond(all_fast, _run_fast, _run_exact,
                              (flat, wt2, a, b))

    keys = mins[:, 0].reshape(BATCH, TOKENS_PER_ROW)
    enc0 = jnp.broadcast_to(args[:TOKENS_PER_ROW, 0][None, :],
                            (BATCH, TOKENS_PER_ROW))

    final_idx = pl.pallas_call(
        _sort_body,
        in_specs=[
            pl.BlockSpec((BATCH, TOKENS_PER_ROW), lambda: (0, 0)),
            pl.BlockSpec((BATCH, TOKENS_PER_ROW), lambda: (0, 0)),
        ],
        out_specs=pl.BlockSpec((BATCH, TOKENS_PER_ROW), lambda: (0, 0)),
        out_shape=jax.ShapeDtypeStruct((BATCH, TOKENS_PER_ROW), jnp.int32),
    )(keys, enc0)

    gathered = _sc_gather(weight, final_idx.reshape(1, N_TOKENS), N_TOKENS)
    return gathered.reshape(input_shape)


# speedup vs baseline: 1.1744x; 1.1020x over previous
import jax
import jax.numpy as jnp
from jax.experimental import pallas as pl
from jax.experimental.pallas import tpu as pltpu
from jax.experimental.pallas import tpu_sc as plsc

NUM_K = 8192
DIM = 64
BATCH = 8
TOKENS_PER_ROW = 1024
N_TOKENS = BATCH * TOKENS_PER_ROW
KBLK = 2048
TBLK = 2048


def _block_argmin(dist, bmin, k):
    iota = jax.lax.broadcasted_iota(
        jnp.int32, dist.shape, 1).astype(jnp.float32)
    barg_f = jnp.min(jnp.where(dist == bmin, iota, jnp.float32(2.0 ** 30)),
                     axis=1, keepdims=True)
    return barg_f.astype(jnp.int32) + k * KBLK


def _fast_body(u_ref, wt2_ref, a_ref, min_ref, arg_ref):
    t = pl.program_id(0)
    k = pl.program_id(1)
    nk = pl.num_programs(1)
    c2 = jax.lax.dot_general(
        u_ref[...], wt2_ref[...], (((1,), (0,)), ((), ())),
        preferred_element_type=jnp.float32)
    a = a_ref[...]
    bmax = jnp.max(c2, axis=1, keepdims=True)

    @pl.when(t == 0)
    def _():
        bmin_d = a - bmax
        dist = a - c2
        barg = _block_argmin(dist, bmin_d, k)

        @pl.when(k == 0)
        def _():
            arg_ref[...] = barg

        @pl.when(k != 0)
        def _():
            run_d = a - min_ref[...]
            arg_ref[...] = jnp.where(bmin_d < run_d, barg, arg_ref[...])

    @pl.when(k == 0)
    def _():
        min_ref[...] = bmax

    @pl.when(k != 0)
    def _():
        min_ref[...] = jnp.maximum(min_ref[...], bmax)

    @pl.when(k == nk - 1)
    def _():
        min_ref[...] = a - min_ref[...]


def _exact_body(u_ref, wt2_ref, a_ref, b_ref, min_ref, arg_ref):
    t = pl.program_id(0)
    k = pl.program_id(1)
    c2 = jax.lax.dot_general(
        u_ref[...], wt2_ref[...], (((1,), (0,)), ((), ())),
        preferred_element_type=jnp.float32)
    dist = (a_ref[...] + b_ref[...]) - c2
    bmin = jnp.min(dist, axis=1, keepdims=True)

    @pl.when(t == 0)
    def _():
        barg = _block_argmin(dist, bmin, k)

        @pl.when(k == 0)
        def _():
            arg_ref[...] = barg

        @pl.when(k != 0)
        def _():
            arg_ref[...] = jnp.where(bmin < min_ref[...], barg, arg_ref[...])

    @pl.when(k == 0)
    def _():
        min_ref[...] = bmin

    @pl.when(k != 0)
    def _():
        min_ref[...] = jnp.minimum(min_ref[...], bmin)


def _sort_body(key_ref, enc_ref, out_ref):
    keys = key_ref[...]
    vals = enc_ref[...]
    n = keys.shape[1]
    pos = jax.lax.broadcasted_iota(jnp.int32, keys.shape, 1)
    i = pos

    def partner(x, j, lower):
        fwd = jnp.roll(x, -j, axis=1)
        bwd = jnp.roll(x, j, axis=1)
        return jnp.where(lower, fwd, bwd)

    k = 2
    while k <= n:
        j = k // 2
        while j >= 1:
            lower = (i & j) == 0
            pk = partner(keys, j, lower)
            pp = partner(pos, j, lower)
            pv = partner(vals, j, lower)
            asc = (i & k) == 0
            lt = (pk < keys) | ((pk == keys) & (pp < pos))
            take = lt == (lower == asc)
            keys = jnp.where(take, pk, keys)
            pos = jnp.where(take, pp, pos)
            vals = jnp.where(take, pv, vals)
            j //= 2
        k *= 2
    out_ref[...] = vals


def _sc_gather(weight, idx2d, n_rows):
    mesh = plsc.VectorSubcoreMesh(core_axis_name="core",
                                  subcore_axis_name="subcore")
    window = 256
    wpad = jnp.pad(weight, ((0, 0), (0, 128 - DIM)))

    @pl.kernel(out_type=jax.ShapeDtypeStruct((n_rows, 128), weight.dtype),
               mesh=mesh)
    def kern(x_hbm, i_hbm, o_hbm):
        def body(i_vmem, o_vmem):
            pltpu.sync_copy(x_hbm.at[i_vmem.at[0]], o_vmem)

        pltpu.emit_pipeline(
            body,
            grid=(n_rows // window,),
            in_specs=[pl.BlockSpec((1, window), lambda i: (0, i))],
            out_specs=[pl.BlockSpec((window, 128), lambda i: (i, 0))],
            core_axis_name=("core", "subcore"),
            dimension_semantics=(pltpu.PARALLEL,),
        )(i_hbm, o_hbm)

    return kern(wpad, idx2d)[:, :DIM]


def kernel(inputs, weight):
    input_shape = inputs.shape
    flat = inputs.reshape(-1, DIM)
    wt2 = 2.0 * weight.T
    a = jnp.sum(flat ** 2, axis=1, keepdims=True)
    b = jnp.sum(weight ** 2, axis=1)[None, :]
    all_fast = (jnp.min(a) >= 16.0) & (jnp.max(b) < 2.0 ** -20)

    grid = (N_TOKENS // TBLK, NUM_K // KBLK)
    out_specs = [
        pl.BlockSpec((TBLK, 1), lambda t, k: (t, 0)),
        pl.BlockSpec((TBLK, 1), lambda t, k: (0, 0)),
    ]
    out_shape = [
        jax.ShapeDtypeStruct((N_TOKENS, 1), jnp.float32),
        jax.ShapeDtypeStruct((TBLK, 1), jnp.int32),
    ]
    cparams = pltpu.CompilerParams(
        dimension_semantics=("parallel", "arbitrary"))

    def _run_fast(ops):
        flat, wt2, a, _ = ops
        return pl.pallas_call(
            _fast_body,
            grid=grid,
            in_specs=[
                pl.BlockSpec((TBLK, DIM), lambda t, k: (t, 0)),
                pl.BlockSpec((DIM, KBLK), lambda t, k: (0, k)),
                pl.BlockSpec((TBLK, 1), lambda t, k: (t, 0)),
            ],
            out_specs=out_specs,
            out_shape=out_shape,
            compiler_params=cparams,
        )(flat, wt2, a)

    def _run_exact(ops):
        flat, wt2, a, b = ops
        return pl.pallas_call(
            _exact_body,
            grid=grid,
            in_specs=[
                pl.BlockSpec((TBLK, DIM), lambda t, k: (t, 0)),
                pl.BlockSpec((DIM, KBLK), lambda t, k: (0, k)),
                pl.BlockSpec((TBLK, 1), lambda t, k: (t, 0)),
                pl.BlockSpec((1, KBLK), lambda t, k: (0, k)),
            ],
            out_specs=out_specs,
            out_shape=out_shape,
            compiler_params=cparams,
        )(flat, wt2, a, b)

    mins, args = jax.lax.cond(all_fast, _run_fast, _run_exact,
                              (flat, wt2, a, b))

    keys = mins[:, 0].reshape(BATCH, TOKENS_PER_ROW)
    enc0 = jnp.broadcast_to(args[:TOKENS_PER_ROW, 0][None, :],
                            (BATCH, TOKENS_PER_ROW))

    final_idx = pl.pallas_call(
        _sort_body,
        in_specs=[
            pl.BlockSpec((BATCH, TOKENS_PER_ROW), lambda: (0, 0)),
            pl.BlockSpec((BATCH, TOKENS_PER_ROW), lambda: (0, 0)),
        ],
        out_specs=pl.BlockSpec((BATCH, TOKENS_PER_ROW), lambda: (0, 0)),
        out_shape=jax.ShapeDtypeStruct((BATCH, TOKENS_PER_ROW), jnp.int32),
    )(keys, enc0)

    gathered = _sc_gather(weight, final_idx.reshape(1, N_TOKENS), N_TOKENS)
    return gathered.reshape(input_shape)
